# baseline (speedup 1.0000x reference)
import numpy as np
import jax
import jax.numpy as jnp
from jax.experimental import pallas as pl
from jax.experimental.pallas import tpu as pltpu

_DIM = 768
_HEADS = 12
_HD = _DIM // _HEADS
_R = 4
_WIN = 4
_ITERS = 2
_L = 1024
_K = 1024
_EH = 16
_NO = 2 * _WIN
_GR = _HEADS * _R
_LAMW = _GR * _HD

_F32 = jnp.float32
_BF16 = jnp.bfloat16


def _rope_tables():
    inv = 1.0 / (10000.0 ** (np.arange(0, _HD, 2, dtype=np.float64) / _HD))
    cos_rows, sin_rows = [], []
    for o in [o for o in range(-_WIN, _WIN + 1) if o != 0]:
        ang = float(-o) * inv
        cos = np.concatenate([np.cos(ang), np.cos(ang)])
        sin = np.concatenate([np.sin(ang), np.sin(ang)])
        cos_rows.append(np.tile(cos, _HEADS))
        sin_rows.append(np.tile(sin, _HEADS))
    return (np.asarray(cos_rows, np.float32), np.asarray(sin_rows, np.float32))

_COS_TAB, _SIN_TAB = _rope_tables()
_S1 = np.kron(np.eye(_GR, dtype=np.float32), np.ones((_HD, 1), np.float32))


def _softplus(x):
    return jnp.maximum(x, 0.0) + jnp.log1p(jnp.exp(-jnp.abs(x)))


def _gelu(x):
    return 0.5 * x * (1.0 + jax.lax.erf(x * np.float32(1.0 / np.sqrt(2.0))))


def _rot_half(u):
    parts = []
    for h in range(_HEADS):
        b = h * _HD
        parts.append(-u[:, b + _HD // 2 : b + _HD])
        parts.append(u[:, b : b + _HD // 2])
    return jnp.concatenate(parts, axis=1)


def _rep4(x):
    reps = []
    for h in range(_HEADS):
        xh = x[:, h * _HD : (h + 1) * _HD]
        reps += [xh, xh, xh, xh]
    return jnp.concatenate(reps, axis=1)


def _rsum4(q):
    parts = []
    for h in range(_HEADS):
        b = h * _R * _HD
        parts.append(q[:, b : b + _HD] + q[:, b + _HD : b + 2 * _HD]
                     + q[:, b + 2 * _HD : b + 3 * _HD]
                     + q[:, b + 3 * _HD : b + 4 * _HD])
    return jnp.concatenate(parts, axis=1)


def _encode_body(t_ref, c_ref, Wt_ref, bt_ref, Wc_ref, bc_ref,
                 aW1t_ref, aW1c_ref, lW1t_ref, lW1c_ref,
                 u_ref, v_ref, At_ref, Ac_ref, Lt_ref, Lc_ref):
    t = t_ref[0].astype(_BF16)
    c = c_ref[0].astype(_BF16)
    te = jnp.dot(t, Wt_ref[...], preferred_element_type=_F32) + bt_ref[...]
    ce = jnp.dot(c, Wc_ref[...], preferred_element_type=_F32) + bc_ref[...]
    u_ref[0] = te
    v_ref[0] = ce
    te_bf = te.astype(_BF16)
    ce_bf = ce.astype(_BF16)
    At_ref[0] = jnp.dot(te_bf, aW1t_ref[...], preferred_element_type=_F32)
    Ac_ref[0] = jnp.dot(ce_bf, aW1c_ref[...], preferred_element_type=_F32)
    Lt_ref[0] = jnp.dot(te_bf, lW1t_ref[...], preferred_element_type=_F32)
    Lc_ref[0] = jnp.dot(ce_bf, lW1c_ref[...], preferred_element_type=_F32)


def _prep_body(ALsh_ref, At_ref, Lt_ref,
               ab1_ref, aW2_ref, ab2_ref, lb1_ref, lW2_ref, lb2_ref,
               s1_ref, lam_ref, invn2_ref, alpha_ref):
    ALsh = ALsh_ref[0, 0]
    Ash = ALsh[:, :_EH].astype(_F32)
    Lsh = ALsh[:, _EH:].astype(_F32)

    ha = _gelu(At_ref[0] + Ash + ab1_ref[...])
    alpha_ref[0, 0] = _softplus(
        jnp.sum(ha * aW2_ref[...], axis=-1, keepdims=True)
        + ab2_ref[...].reshape(1, 1))
    G = _gelu(Lt_ref[0] + Lsh + lb1_ref[...])
    lam_bf = (jnp.dot(G.astype(_BF16), lW2_ref[...],
                      preferred_element_type=_F32)
              + lb2_ref[...]).astype(_BF16)
    lam_ref[0, 0] = lam_bf
    n2 = jnp.dot(lam_bf * lam_bf, s1_ref[...], preferred_element_type=_F32)
    invn2_ref[0, 0] = 1.0 / jnp.maximum(n2, 1e-24)


def _iter_body(u_ref, vsh_ref, lam_ref, invn2_ref, alpha_ref,
               eta_ref, cos_ref, sin_ref, s1_ref, s1t_ref,
               unew_ref, vu_ref):
    io = pl.program_id(1)
    o = jnp.where(io < _WIN, io - _WIN, io - (_WIN - 1))

    u = u_ref[0]
    vsh = vsh_ref[0, 0]
    lam_bf = lam_ref[0, 0]
    invn2 = invn2_ref[0, 0]
    alpha = alpha_ref[0, 0]

    rot = _rot_half(u)
    roped = u * cos_ref[0] + rot * sin_ref[0]
    diff = roped - vsh

    drep = _rep4(diff.astype(_BF16))
    ld = jnp.dot(lam_bf * drep, s1_ref[...], preferred_element_type=_F32)
    q = (ld * invn2).astype(_BF16)
    qrep = jnp.dot(q, s1t_ref[...],
                   preferred_element_type=_F32).astype(_BF16)
    resid2 = _rsum4(qrep * lam_bf)
    resid = alpha * diff + resid2.astype(_F32)

    rows = jax.lax.broadcasted_iota(jnp.int32, (_L, 1), 0)
    valid = (rows + o >= 0) & (rows + o < _K)
    resid = jnp.where(valid, resid, 0.0)

    @pl.when(io == 0)
    def _():
        vu_ref[...] = resid

    @pl.when(io > 0)
    def _():
        vu_ref[...] += resid

    @pl.when(io == _NO - 1)
    def _():
        eta = _softplus(eta_ref[...])
        unew_ref[0] = u - eta * vu_ref[...]


def _proj_body(t_ref, u_ref, oW_ref, ob_ref, out_ref):
    out_ref[0] = (t_ref[0]
                  + jnp.dot(u_ref[0].astype(_BF16), oW_ref[...],
                            preferred_element_type=_F32)
                  + ob_ref[...])


def _full(shape):
    return pl.BlockSpec(shape, lambda *a: (0,) * len(shape))


def _bblk(shape):
    return pl.BlockSpec((1,) + shape, lambda b, *a: (b,) + (0,) * len(shape))


def _oblk(shape):
    return pl.BlockSpec((1, 1) + shape, lambda b, io: (b, io) + (0,) * len(shape))


def kernel(target, context, Wt, bt, Wc, bc, aW1, ab1, aW2, ab2,
           lW1, lb1, lW2, lb2, step_sizes, oW, ob):
    B, L, dim = target.shape
    assert (B, L, dim) == (2, _L, _DIM) and context.shape == (B, _K, _DIM)
    f32 = jnp.float32

    enc_out = pl.pallas_call(
        _encode_body,
        grid=(B,),
        in_specs=[_bblk((_L, _DIM)), _bblk((_K, _DIM)),
                  _full((_DIM, _DIM)), _full((_DIM,)),
                  _full((_DIM, _DIM)), _full((_DIM,)),
                  _full((_DIM, _EH)), _full((_DIM, _EH)),
                  _full((_DIM, _EH)), _full((_DIM, _EH))],
        out_specs=[_bblk((_L, _DIM)), _bblk((_K, _DIM)),
                   _bblk((_L, _EH)), _bblk((_K, _EH)),
                   _bblk((_L, _EH)), _bblk((_K, _EH))],
        out_shape=[jax.ShapeDtypeStruct((B, _L, _DIM), f32),
                   jax.ShapeDtypeStruct((B, _K, _DIM), f32),
                   jax.ShapeDtypeStruct((B, _L, _EH), f32),
                   jax.ShapeDtypeStruct((B, _K, _EH), f32),
                   jax.ShapeDtypeStruct((B, _L, _EH), f32),
                   jax.ShapeDtypeStruct((B, _K, _EH), f32)],
    )(target, context, Wt.astype(_BF16), bt, Wc.astype(_BF16), bc,
      aW1[:_DIM].astype(_BF16), aW1[_DIM:].astype(_BF16),
      lW1[:_DIM].astype(_BF16), lW1[_DIM:].astype(_BF16))
    u, v, At, Ac, Lt, Lc = enc_out

    offs = [o for o in range(-_WIN, _WIN + 1) if o != 0]
    v_bf = v.astype(_BF16)
    AL = jnp.concatenate([Ac, Lc], axis=-1).astype(_BF16)
    vsh_all = jnp.stack([jnp.roll(v_bf, -o, axis=1) for o in offs], axis=1)
    ALsh_all = jnp.stack([jnp.roll(AL, -o, axis=1) for o in offs], axis=1)

    lW2_bf = lW2.astype(_BF16)
    s1 = jnp.asarray(_S1, _BF16)
    s1t = jnp.asarray(_S1.T, _BF16)
    cos_tab = jnp.asarray(_COS_TAB).reshape(_NO, 1, _DIM)
    sin_tab = jnp.asarray(_SIN_TAB).reshape(_NO, 1, _DIM)

    lam_all, invn2_all, alpha_all = pl.pallas_call(
        _prep_body,
        grid=(B, _NO),
        in_specs=[_oblk((_K, 2 * _EH)),
                  _bblk((_L, _EH)), _bblk((_L, _EH)),
                  _full((_EH,)), _full((1, _EH)), _full((1,)),
                  _full((_EH,)), _full((_EH, _LAMW)), _full((_LAMW,)),
                  _full((_LAMW, _GR))],
        out_specs=[_oblk((_L, _LAMW)), _oblk((_L, _GR)), _oblk((_L, 1))],
        out_shape=[jax.ShapeDtypeStruct((B, _NO, _L, _LAMW), _BF16),
                   jax.ShapeDtypeStruct((B, _NO, _L, _GR), f32),
                   jax.ShapeDtypeStruct((B, _NO, _L, 1), f32)],
    )(ALsh_all, At, Lt, ab1, aW2.T, ab2, lb1, lW2_bf, lb2, s1)

    iter_call = pl.pallas_call(
        _iter_body,
        grid=(B, _NO),
        in_specs=[_bblk((_L, _DIM)),
                  _oblk((_K, _DIM)), _oblk((_L, _LAMW)),
                  _oblk((_L, _GR)), _oblk((_L, 1)),
                  _full((_L, 1)),
                  pl.BlockSpec((1, 1, _DIM), lambda b, io: (io, 0, 0)),
                  pl.BlockSpec((1, 1, _DIM), lambda b, io: (io, 0, 0)),
                  _full((_LAMW, _GR)), _full((_GR, _LAMW))],
        out_specs=_bblk((_L, _DIM)),
        out_shape=jax.ShapeDtypeStruct((B, _L, _DIM), f32),
        scratch_shapes=[pltpu.VMEM((_L, _DIM), f32)],
    )
    for t_it in range(_ITERS):
        eta_col = step_sizes[t_it, :_L].reshape(_L, 1)
        u = iter_call(u, vsh_all, lam_all, invn2_all, alpha_all,
                      eta_col, cos_tab, sin_tab, s1, s1t)

    return pl.pallas_call(
        _proj_body,
        grid=(B,),
        in_specs=[_bblk((_L, _DIM)), _bblk((_L, _DIM)),
                  _full((_DIM, _DIM)), _full((_DIM,))],
        out_specs=_bblk((_L, _DIM)),
        out_shape=jax.ShapeDtypeStruct((B, L, dim), f32),
    )(target, u, oW.astype(_BF16), ob)


# speedup vs baseline: 1.2302x; 1.2302x over previous
import numpy as np
import jax
import jax.numpy as jnp
from jax.experimental import pallas as pl
from jax.experimental.pallas import tpu as pltpu

_DIM = 768
_HEADS = 12
_HD = _DIM // _HEADS
_R = 4
_WIN = 4
_ITERS = 2
_L = 1024
_K = 1024
_EH = 16
_NO = 2 * _WIN
_GR = _HEADS * _R
_LAMW = _GR * _HD

_F32 = jnp.float32
_BF16 = jnp.bfloat16


def _rope_tables():
    inv = 1.0 / (10000.0 ** (np.arange(0, _HD, 2, dtype=np.float64) / _HD))
    cos_rows, sin_rows = [], []
    for o in [o for o in range(-_WIN, _WIN + 1) if o != 0]:
        ang = float(-o) * inv
        cos = np.concatenate([np.cos(ang), np.cos(ang)])
        sin = np.concatenate([np.sin(ang), np.sin(ang)])
        cos_rows.append(np.tile(cos, _HEADS))
        sin_rows.append(np.tile(sin, _HEADS))
    return (np.asarray(cos_rows, np.float32), np.asarray(sin_rows, np.float32))

_COS_TAB, _SIN_TAB = _rope_tables()
_S1 = np.kron(np.eye(_GR, dtype=np.float32), np.ones((_HD, 1), np.float32))


def _softplus(x):
    return jnp.maximum(x, 0.0) + jnp.log1p(jnp.exp(-jnp.abs(x)))


def _gelu(x):
    return 0.5 * x * (1.0 + jax.lax.erf(x * np.float32(1.0 / np.sqrt(2.0))))


def _rot_half(u):
    parts = []
    for h in range(_HEADS):
        b = h * _HD
        parts.append(-u[:, b + _HD // 2 : b + _HD])
        parts.append(u[:, b : b + _HD // 2])
    return jnp.concatenate(parts, axis=1)


def _rep4(x):
    reps = []
    for h in range(_HEADS):
        xh = x[:, h * _HD : (h + 1) * _HD]
        reps += [xh, xh, xh, xh]
    return jnp.concatenate(reps, axis=1)


def _rsum4(q):
    parts = []
    for h in range(_HEADS):
        b = h * _R * _HD
        parts.append(q[:, b : b + _HD] + q[:, b + _HD : b + 2 * _HD]
                     + q[:, b + 2 * _HD : b + 3 * _HD]
                     + q[:, b + 3 * _HD : b + 4 * _HD])
    return jnp.concatenate(parts, axis=1)


def _encode_body(t_ref, c_ref, Wt_ref, bt_ref, Wc_ref, bc_ref,
                 aW1t_ref, aW1c_ref, lW1t_ref, lW1c_ref,
                 u_ref, At_ref, Lt_ref, sh_ref):
    t = t_ref[0].astype(_BF16)
    c = c_ref[0].astype(_BF16)
    te = jnp.dot(t, Wt_ref[...], preferred_element_type=_F32) + bt_ref[...]
    ce = jnp.dot(c, Wc_ref[...], preferred_element_type=_F32) + bc_ref[...]
    u_ref[0] = te
    te_bf = te.astype(_BF16)
    ce_bf = ce.astype(_BF16)
    At_ref[0] = jnp.dot(te_bf, aW1t_ref[...], preferred_element_type=_F32)
    Ac = jnp.dot(ce_bf, aW1c_ref[...], preferred_element_type=_F32)
    Lt_ref[0] = jnp.dot(te_bf, lW1t_ref[...], preferred_element_type=_F32)
    Lc = jnp.dot(ce_bf, lW1c_ref[...], preferred_element_type=_F32)
    packed = jnp.concatenate(
        [ce, Ac, Lc], axis=1).astype(_BF16)
    for io, o in enumerate([o for o in range(-_WIN, _WIN + 1) if o != 0]):
        sh_ref[0, io] = jnp.concatenate([packed[o:], packed[:o]], axis=0)


def _iter_body(u_ref, sh_ref, At_ref, Lt_ref,
               ab1_ref, aW2_ref, ab2_ref, lb1_ref, lW2_ref, lb2_ref,
               eta_ref, cos_ref, sin_ref, s1_ref, s1t_ref,
               unew_ref, vu_ref):
    io = pl.program_id(1)
    o = jnp.where(io < _WIN, io - _WIN, io - (_WIN - 1))

    u = u_ref[0]
    sh = sh_ref[0, 0]
    vsh = sh[:, :_DIM]
    Ash = sh[:, _DIM : _DIM + _EH].astype(_F32)
    Lsh = sh[:, _DIM + _EH :].astype(_F32)

    ha = _gelu(At_ref[0] + Ash + ab1_ref[...])
    alpha = _softplus(
        jnp.sum(ha * aW2_ref[...], axis=-1, keepdims=True)
        + ab2_ref[...].reshape(1, 1))
    G = _gelu(Lt_ref[0] + Lsh + lb1_ref[...])
    lam_bf = (jnp.dot(G.astype(_BF16), lW2_ref[...],
                      preferred_element_type=_F32)
              + lb2_ref[...]).astype(_BF16)
    n2 = jnp.dot(lam_bf * lam_bf, s1_ref[...], preferred_element_type=_F32)
    invn2 = 1.0 / jnp.maximum(n2, 1e-24)

    rot = _rot_half(u)
    roped = u * cos_ref[0] + rot * sin_ref[0]
    diff = roped - vsh

    drep = _rep4(diff.astype(_BF16))
    ld = jnp.dot(lam_bf * drep, s1_ref[...], preferred_element_type=_F32)
    q = (ld * invn2).astype(_BF16)
    qrep = jnp.dot(q, s1t_ref[...],
                   preferred_element_type=_F32).astype(_BF16)
    resid2 = _rsum4(qrep * lam_bf)
    resid = alpha * diff + resid2.astype(_F32)

    rows = jax.lax.broadcasted_iota(jnp.int32, (_L, 1), 0)
    valid = (rows + o >= 0) & (rows + o < _K)
    resid = jnp.where(valid, resid, 0.0)

    @pl.when(io == 0)
    def _():
        vu_ref[...] = resid

    @pl.when(io > 0)
    def _():
        vu_ref[...] += resid

    @pl.when(io == _NO - 1)
    def _():
        eta = _softplus(eta_ref[...])
        unew_ref[0] = u - eta * vu_ref[...]


def _proj_body(t_ref, u_ref, oW_ref, ob_ref, out_ref):
    out_ref[0] = (t_ref[0]
                  + jnp.dot(u_ref[0].astype(_BF16), oW_ref[...],
                            preferred_element_type=_F32)
                  + ob_ref[...])


def _full(shape):
    return pl.BlockSpec(shape, lambda *a: (0,) * len(shape))


def _bblk(shape):
    return pl.BlockSpec((1,) + shape, lambda b, *a: (b,) + (0,) * len(shape))


def kernel(target, context, Wt, bt, Wc, bc, aW1, ab1, aW2, ab2,
           lW1, lb1, lW2, lb2, step_sizes, oW, ob):
    B, L, dim = target.shape
    assert (B, L, dim) == (2, _L, _DIM) and context.shape == (B, _K, _DIM)
    f32 = jnp.float32

    enc_out = pl.pallas_call(
        _encode_body,
        grid=(B,),
        in_specs=[_bblk((_L, _DIM)), _bblk((_K, _DIM)),
                  _full((_DIM, _DIM)), _full((_DIM,)),
                  _full((_DIM, _DIM)), _full((_DIM,)),
                  _full((_DIM, _EH)), _full((_DIM, _EH)),
                  _full((_DIM, _EH)), _full((_DIM, _EH))],
        out_specs=[_bblk((_L, _DIM)),
                   _bblk((_L, _EH)), _bblk((_L, _EH)),
                   _bblk((_NO, _K, _DIM + 2 * _EH))],
        out_shape=[jax.ShapeDtypeStruct((B, _L, _DIM), f32),
                   jax.ShapeDtypeStruct((B, _L, _EH), f32),
                   jax.ShapeDtypeStruct((B, _L, _EH), f32),
                   jax.ShapeDtypeStruct((B, _NO, _K, _DIM + 2 * _EH), _BF16)],
    )(target, context, Wt.astype(_BF16), bt, Wc.astype(_BF16), bc,
      aW1[:_DIM].astype(_BF16), aW1[_DIM:].astype(_BF16),
      lW1[:_DIM].astype(_BF16), lW1[_DIM:].astype(_BF16))
    u, At, Lt, sh_all = enc_out

    lW2_bf = lW2.astype(_BF16)
    s1 = jnp.asarray(_S1, _BF16)
    s1t = jnp.asarray(_S1.T, _BF16)
    cos_tab = jnp.asarray(_COS_TAB).reshape(_NO, 1, _DIM)
    sin_tab = jnp.asarray(_SIN_TAB).reshape(_NO, 1, _DIM)

    iter_call = pl.pallas_call(
        _iter_body,
        grid=(B, _NO),
        in_specs=[_bblk((_L, _DIM)),
                  pl.BlockSpec((1, 1, _K, _DIM + 2 * _EH),
                               lambda b, io: (b, io, 0, 0)),
                  _bblk((_L, _EH)), _bblk((_L, _EH)),
                  _full((_EH,)), _full((1, _EH)), _full((1,)),
                  _full((_EH,)), _full((_EH, _LAMW)), _full((_LAMW,)),
                  _full((_L, 1)),
                  pl.BlockSpec((1, 1, _DIM), lambda b, io: (io, 0, 0)),
                  pl.BlockSpec((1, 1, _DIM), lambda b, io: (io, 0, 0)),
                  _full((_LAMW, _GR)), _full((_GR, _LAMW))],
        out_specs=_bblk((_L, _DIM)),
        out_shape=jax.ShapeDtypeStruct((B, _L, _DIM), f32),
        scratch_shapes=[pltpu.VMEM((_L, _DIM), f32)],
    )
    for t_it in range(_ITERS):
        eta_col = step_sizes[t_it, :_L].reshape(_L, 1)
        u = iter_call(u, sh_all, At, Lt, ab1, aW2.T, ab2, lb1,
                      lW2_bf, lb2, eta_col, cos_tab, sin_tab, s1, s1t)

    return pl.pallas_call(
        _proj_body,
        grid=(B,),
        in_specs=[_bblk((_L, _DIM)), _bblk((_L, _DIM)),
                  _full((_DIM, _DIM)), _full((_DIM,))],
        out_specs=_bblk((_L, _DIM)),
        out_shape=jax.ShapeDtypeStruct((B, L, dim), f32),
    )(target, u, oW.astype(_BF16), ob)
